# baseline (speedup 1.0000x reference)
import functools

import jax
import jax.numpy as jnp
from jax import lax
from jax.experimental import pallas as pl
from jax.experimental.pallas import tpu as pltpu
from jax.experimental.pallas import tpu_sc as plsc

VOCAB = 1000000
HIDDEN = 128
B = 1024
L = 200
EPS = 1e-12

NC = 2
NS = 16
LANES = 16
NW = NC * NS
SEQ_PER_W = B // NW
NVEC = HIDDEN // LANES


def _xlane_sum(x):
    lanes = lax.iota(jnp.int32, LANES)
    for k in (8, 4, 2, 1):
        x = x + x.at[lanes ^ k].get(mode="promise_in_bounds")
    return x


def _rsqrt(v):
    vi = lax.bitcast_convert_type(v, jnp.int32)
    y = lax.bitcast_convert_type(jnp.int32(0x5F3759DF) - (vi >> 1),
                                 jnp.float32)
    half = jnp.float32(0.5) * v
    for _ in range(3):
        y = y * (jnp.float32(1.5) - half * y * y)
    return y


def _body(ids_hbm, tok_hbm, pos_hbm, g_hbm, b_hbm, out_hbm,
          pos_v, g_v, b_v, idx_v, rows_v, sem):
    wid = lax.axis_index("s") * NC + lax.axis_index("c")

    pltpu.sync_copy(pos_hbm.at[pl.ds(0, L)], pos_v)
    pltpu.sync_copy(g_hbm, g_v)
    pltpu.sync_copy(b_hbm, b_v)

    gvs = [g_v[pl.ds(c * LANES, LANES)] for c in range(NVEC)]
    bvs = [b_v[pl.ds(c * LANES, LANES)] for c in range(NVEC)]

    def seq_body(s, _):
        seq = wid * SEQ_PER_W + s
        base = seq * L
        pltpu.sync_copy(ids_hbm.at[pl.ds(seq * 2, 2)], idx_v)
        cp0 = pltpu.async_copy(
            tok_hbm.at[idx_v.at[0]], rows_v.at[pl.ds(0, 100)], sem)
        cp1 = pltpu.async_copy(
            tok_hbm.at[idx_v.at[1]], rows_v.at[pl.ds(100, 100)], sem)
        cp0.wait()
        cp1.wait()

        def row_body(r, _):
            xs = []
            ssum = jnp.zeros((LANES,), jnp.float32)
            ssq = jnp.zeros((LANES,), jnp.float32)
            for c in range(NVEC):
                x = (rows_v[r, pl.ds(c * LANES, LANES)]
                     + pos_v[r, pl.ds(c * LANES, LANES)])
                xs.append(x)
                ssum = ssum + x
                ssq = ssq + x * x
            mvec = _xlane_sum(ssum) * jnp.float32(1.0 / HIDDEN)
            var = _xlane_sum(ssq) * jnp.float32(1.0 / HIDDEN) - mvec * mvec
            rinv = _rsqrt(var + jnp.float32(EPS))
            for c in range(NVEC):
                out = (xs[c] - mvec) * rinv * gvs[c] + bvs[c]
                rows_v[r, pl.ds(c * LANES, LANES)] = out
            return 0

        lax.fori_loop(0, L, row_body, 0)
        pltpu.sync_copy(rows_v, out_hbm.at[pl.ds(base, L)])
        return 0

    lax.fori_loop(0, SEQ_PER_W, seq_body, 0)


@jax.jit
def _run(ids2, token_table, pos_table, ln_gamma, ln_beta):
    mesh = plsc.VectorSubcoreMesh(
        core_axis_name="c", subcore_axis_name="s",
        num_cores=NC, num_subcores=NS)
    f = pl.kernel(
        _body,
        out_type=jax.ShapeDtypeStruct((B * L, HIDDEN), jnp.float32),
        mesh=mesh,
        scratch_types=[
            pltpu.VMEM((L, HIDDEN), jnp.float32),
            pltpu.VMEM((HIDDEN,), jnp.float32),
            pltpu.VMEM((HIDDEN,), jnp.float32),
            pltpu.VMEM((2, 100), jnp.int32),
            pltpu.VMEM((L, HIDDEN), jnp.float32),
            pltpu.SemaphoreType.DMA,
        ],
    )
    return f(ids2, token_table, pos_table, ln_gamma, ln_beta)


def kernel(input_ids, token_table, pos_table, ln_gamma, ln_beta):
    ids2 = input_ids.reshape(B * L // 100, 100)
    out = _run(ids2, token_table, pos_table, ln_gamma, ln_beta)
    return out.reshape(B, L, HIDDEN)


# speedup vs baseline: 1.3630x; 1.3630x over previous
import functools

import jax
import jax.numpy as jnp
from jax import lax
from jax.experimental import pallas as pl
from jax.experimental.pallas import tpu as pltpu
from jax.experimental.pallas import tpu_sc as plsc

VOCAB = 1000000
HIDDEN = 128
B = 1024
L = 200
EPS = 1e-12

NC = 2
NS = 16
LANES = 16
NW = NC * NS
SEQ_PER_W = B // NW
NVEC = HIDDEN // LANES


def _xlane_sum(x):
    lanes = lax.iota(jnp.int32, LANES)
    for k in (8, 4, 2, 1):
        x = x + x.at[lanes ^ k].get(mode="promise_in_bounds")
    return x


def _rsqrt(v):
    vi = lax.bitcast_convert_type(v, jnp.int32)
    y = lax.bitcast_convert_type(jnp.int32(0x5F3759DF) - (vi >> 1),
                                 jnp.float32)
    half = jnp.float32(0.5) * v
    for _ in range(3):
        y = y * (jnp.float32(1.5) - half * y * y)
    return y


def _rsqrt2(v):
    vi = lax.bitcast_convert_type(v, jnp.int32)
    y = lax.bitcast_convert_type(jnp.int32(0x5F3759DF) - (vi >> 1),
                                 jnp.float32)
    half = jnp.float32(0.5) * v
    for _ in range(2):
        y = y * (jnp.float32(1.5) - half * y * y)
    return y


def _body(ids_hbm, tok_hbm, pos_hbm, g_hbm, b_hbm, out_hbm,
          pos_v, g_v, b_v, idx_v, rows_v, sem):
    wid = lax.axis_index("s") * NC + lax.axis_index("c")

    pltpu.sync_copy(pos_hbm.at[pl.ds(0, L)], pos_v)
    pltpu.sync_copy(g_hbm, g_v)
    pltpu.sync_copy(b_hbm, b_v)

    gvs = [g_v[pl.ds(c * LANES, LANES)] for c in range(NVEC)]
    bvs = [b_v[pl.ds(c * LANES, LANES)] for c in range(NVEC)]

    def seq_body(s, _):
        seq = wid * SEQ_PER_W + s
        base = seq * L
        pltpu.sync_copy(ids_hbm.at[pl.ds(seq * 2, 2)], idx_v)
        cp0 = pltpu.async_copy(
            tok_hbm.at[idx_v.at[0]], rows_v.at[pl.ds(0, 100)], sem)
        cp1 = pltpu.async_copy(
            tok_hbm.at[idx_v.at[1]], rows_v.at[pl.ds(100, 100)], sem)
        cp0.wait()
        cp1.wait()

        @plsc.parallel_loop(0, L, step=1, unroll=4)
        def row_body(r):
            xs = []
            ssum = jnp.zeros((LANES,), jnp.float32)
            ssq = jnp.zeros((LANES,), jnp.float32)
            for c in range(NVEC):
                x = (rows_v[r, pl.ds(c * LANES, LANES)]
                     + pos_v[r, pl.ds(c * LANES, LANES)])
                xs.append(x)
                ssum = ssum + x
                ssq = ssq + x * x
            mvec = _xlane_sum(ssum) * jnp.float32(1.0 / HIDDEN)
            var = _xlane_sum(ssq) * jnp.float32(1.0 / HIDDEN) - mvec * mvec
            rg = _rsqrt2(var + jnp.float32(EPS))
            for c in range(NVEC):
                out = (xs[c] - mvec) * rg * gvs[c] + bvs[c]
                rows_v[r, pl.ds(c * LANES, LANES)] = out
        pltpu.sync_copy(rows_v, out_hbm.at[pl.ds(base, L)])
        return 0

    lax.fori_loop(0, SEQ_PER_W, seq_body, 0)


@jax.jit
def _run(ids2, token_table, pos_table, ln_gamma, ln_beta):
    mesh = plsc.VectorSubcoreMesh(
        core_axis_name="c", subcore_axis_name="s",
        num_cores=NC, num_subcores=NS)
    f = pl.kernel(
        _body,
        out_type=jax.ShapeDtypeStruct((B * L, HIDDEN), jnp.float32),
        mesh=mesh,
        scratch_types=[
            pltpu.VMEM((L, HIDDEN), jnp.float32),
            pltpu.VMEM((HIDDEN,), jnp.float32),
            pltpu.VMEM((HIDDEN,), jnp.float32),
            pltpu.VMEM((2, 100), jnp.int32),
            pltpu.VMEM((L, HIDDEN), jnp.float32),
            pltpu.SemaphoreType.DMA,
        ],
    )
    return f(ids2, token_table, pos_table, ln_gamma, ln_beta)


def kernel(input_ids, token_table, pos_table, ln_gamma, ln_beta):
    ids2 = input_ids.reshape(B * L // 100, 100)
    out = _run(ids2, token_table, pos_table, ln_gamma, ln_beta)
    return out.reshape(B, L, HIDDEN)


# speedup vs baseline: 2.3504x; 1.7244x over previous
import functools

import jax
import jax.numpy as jnp
from jax import lax
from jax.experimental import pallas as pl
from jax.experimental.pallas import tpu as pltpu
from jax.experimental.pallas import tpu_sc as plsc

VOCAB = 1000000
HIDDEN = 128
B = 1024
L = 200
EPS = 1e-12

NC = 2
NS = 16
LANES = 16
NW = NC * NS
ROWS_W = B * L // NW
UR = 128
NU = ROWS_W // UR
NBUF = 5
NVEC = HIDDEN // LANES


def _xlane_sum(x):
    lanes = lax.iota(jnp.int32, LANES)
    for k in (8, 4, 2, 1):
        x = x + x.at[lanes ^ k].get(mode="promise_in_bounds")
    return x


def _rsqrt(v):
    vi = lax.bitcast_convert_type(v, jnp.int32)
    y = lax.bitcast_convert_type(jnp.int32(0x5F3759DF) - (vi >> 1),
                                 jnp.float32)
    half = jnp.float32(0.5) * v
    for _ in range(2):
        y = y * (jnp.float32(1.5) - half * y * y)
    return y


def _body(ids_hbm, tok_hbm, pos_hbm, g_hbm, b_hbm, out_hbm,
          pos_v, g_v, b_v, idx_v, rows_v, gsems, wsems):
    wid = lax.axis_index("s") * NC + lax.axis_index("c")

    pltpu.sync_copy(pos_hbm.at[pl.ds(0, L)], pos_v)
    pltpu.sync_copy(g_hbm, g_v)
    pltpu.sync_copy(b_hbm, b_v)
    pltpu.sync_copy(ids_hbm.at[pl.ds(wid * ROWS_W, ROWS_W)], idx_v)

    gvs = [g_v[pl.ds(c * LANES, LANES)] for c in range(NVEC)]
    bvs = [b_v[pl.ds(c * LANES, LANES)] for c in range(NVEC)]

    def fire_gather(u, b):
        return pltpu.async_copy(
            tok_hbm.at[idx_v.at[pl.ds(u * UR, UR)]], rows_v.at[b], gsems[b])

    for b in range(NBUF - 1):
        fire_gather(b, b)

    def unit_group(g, _):
        for b in range(NBUF):
            u = g + b
            pltpu.make_async_copy(
                tok_hbm.at[idx_v.at[pl.ds(u * UR, UR)]], rows_v.at[b],
                gsems[b]).wait()
            pbase = lax.rem(u * UR, L)

            @plsc.parallel_loop(0, UR, step=1, unroll=4)
            def row_body(r):
                xs = []
                ssum = jnp.zeros((LANES,), jnp.float32)
                ssq = jnp.zeros((LANES,), jnp.float32)
                p = pbase + r
                p = jnp.where(p >= L, p - L, p)
                for c in range(NVEC):
                    x = (rows_v[b, r, pl.ds(c * LANES, LANES)]
                         + pos_v[p, pl.ds(c * LANES, LANES)])
                    xs.append(x)
                    ssum = ssum + x
                    ssq = ssq + x * x
                mvec = _xlane_sum(ssum) * jnp.float32(1.0 / HIDDEN)
                var = (_xlane_sum(ssq) * jnp.float32(1.0 / HIDDEN)
                       - mvec * mvec)
                rg = _rsqrt(var + jnp.float32(EPS))
                for c in range(NVEC):
                    out = (xs[c] - mvec) * rg * gvs[c] + bvs[c]
                    rows_v[b, r, pl.ds(c * LANES, LANES)] = out

            base = wid * ROWS_W + u * UR
            pltpu.async_copy(
                rows_v.at[b], out_hbm.at[pl.ds(base, UR)], wsems[b])

            nb = (b + NBUF - 1) % NBUF

            @pl.when(u >= 1)
            def _wait_prev_wb():
                pltpu.make_async_copy(
                    rows_v.at[nb],
                    out_hbm.at[pl.ds(wid * ROWS_W + (u - 1) * UR, UR)],
                    wsems[nb]).wait()

            @pl.when(u + NBUF - 1 < NU)
            def _refill():
                fire_gather(u + NBUF - 1, nb)

        return 0

    lax.fori_loop(0, NU // NBUF, lambda i, c: unit_group(i * NBUF, c), 0)

    last = NU - 1
    pltpu.make_async_copy(
        rows_v.at[last % NBUF],
        out_hbm.at[pl.ds(wid * ROWS_W + last * UR, UR)],
        wsems[last % NBUF]).wait()


@jax.jit
def _run(ids2, token_table, pos_table, ln_gamma, ln_beta):
    mesh = plsc.VectorSubcoreMesh(
        core_axis_name="c", subcore_axis_name="s",
        num_cores=NC, num_subcores=NS)
    f = pl.kernel(
        _body,
        out_type=jax.ShapeDtypeStruct((B * L, HIDDEN), jnp.float32),
        mesh=mesh,
        scratch_types=[
            pltpu.VMEM((L, HIDDEN), jnp.float32),
            pltpu.VMEM((HIDDEN,), jnp.float32),
            pltpu.VMEM((HIDDEN,), jnp.float32),
            pltpu.VMEM((ROWS_W,), jnp.int32),
            pltpu.VMEM((NBUF, UR, HIDDEN), jnp.float32),
            [pltpu.SemaphoreType.DMA] * NBUF,
            [pltpu.SemaphoreType.DMA] * NBUF,
        ],
    )
    return f(ids2, token_table, pos_table, ln_gamma, ln_beta)


def kernel(input_ids, token_table, pos_table, ln_gamma, ln_beta):
    ids1 = input_ids.reshape(B * L)
    out = _run(ids1, token_table, pos_table, ln_gamma, ln_beta)
    return out.reshape(B, L, HIDDEN)
